# baseline (speedup 1.0000x reference)
import functools

import jax
import jax.numpy as jnp
from jax import lax
from jax.experimental import pallas as pl
from jax.experimental.pallas import tpu as pltpu
from jax.experimental.pallas import tpu_sc as plsc

N = 20000
C = 64
K27 = 27
NPAD = 20480
Q = NPAD // 32
NGRP = N // 16
BASE = 68
TSH = 9840
TPAD = 32 * TSH
SENT = 2**30
NQ = K27 * Q
NQ_PAD = NQ + 128
HP = 14
HROWS = HP * NPAD

_LANE = lambda: lax.broadcasted_iota(jnp.int32, (16,), 0)


def _shift_right_one(v):
    idx = jnp.maximum(_LANE() - 1, 0)
    return lax.gather(
        v, idx[:, None],
        lax.GatherDimensionNumbers(
            offset_dims=(), collapsed_slice_dims=(0,), start_index_map=(0,)),
        slice_sizes=(1,),
        mode=lax.GatherScatterMode.PROMISE_IN_BOUNDS)


def _keys16(posb, g):
    off = g * 16
    x = posb[0, pl.ds(off, 16)].astype(jnp.int32)
    y = posb[1, pl.ds(off, 16)].astype(jnp.int32)
    z = posb[2, pl.ds(off, 16)].astype(jnp.int32)
    return ((x + 2) * BASE + (y + 2)) * BASE + (z + 2)


def _sc_build(pos_hbm, table_hbm, posb, tbl):
    wid = lax.axis_index("s") * 2 + lax.axis_index("c")
    shard_base = wid * TSH
    pltpu.sync_copy(pos_hbm, posb)

    def init_body(g, _):
        tbl[pl.ds(g * 16, 16)] = jnp.full((16,), SENT, jnp.int32)
        return 0
    lax.fori_loop(0, TSH // 16, init_body, 0)

    lane = _LANE()

    def body(g, _):
        key = _keys16(posb, g)
        p = g * 16 + lane
        dup = key != key
        for j in range(15):
            kj = key[j]
            dup = dup | ((lane > j) & (key == kj))
        addr = key - shard_base
        m = ~dup & (addr >= 0) & (addr < TSH)
        addr_c = jnp.clip(addr, 0, TSH - 1)
        cur = plsc.load_gather(tbl, [addr_c], mask=m)
        newv = jnp.minimum(jnp.where(m, cur, SENT), p)
        plsc.store_scatter(tbl, [addr_c], newv, mask=m)
        return 0
    lax.fori_loop(0, NGRP, body, 0)

    pltpu.sync_copy(tbl, table_hbm.at[pl.ds(shard_base, TSH)])


def _sc_conv(pos_hbm, table_hbm, h_hbm, out_hbm,
             posb, qk, lkp, cfx, cab, grows0, grows1, outb,
             sem, sem0, sem1):
    wid = lax.axis_index("s") * 2 + lax.axis_index("c")
    pbase = wid * Q
    pltpu.sync_copy(pos_hbm.at[:, pl.ds(pbase, Q)], posb)
    lane = _LANE()
    scope = jax.named_scope

    def zero_body(g, _):
        outb[pl.ds(g * 16, 16)] = jnp.zeros((16,), jnp.float32)
        return 0
    with scope("ph0_zero"):
        lax.fori_loop(0, Q * C // 16, zero_body, 0)

    def pad_body(g, _):
        cfx[pl.ds(g * 16, 16)] = jnp.full((16,), HROWS - 1, jnp.int32)
        cab[pl.ds(g * 16, 16)] = jnp.full((16,), (Q * C - 1) * 2,
                                          jnp.int32)
        return 0
    with scope("ph0_pad"):
        lax.fori_loop(0, NQ_PAD // 16, pad_body, 0)

    scope1 = scope("ph1_keys")
    scope1.__enter__()
    for i in range(K27):
        dx, dy, dz = i // 9 - 1, (i // 3) % 3 - 1, i % 3 - 1
        doff = (dx * BASE + dy) * BASE + dz

        def q_body(g, _, i=i, doff=doff):
            key = _keys16(posb, g)
            qk[pl.ds(i * Q + g * 16, 16)] = key + doff
            return 0
        lax.fori_loop(0, Q // 16, q_body, 0)

    scope1.__exit__(None, None, None)

    with scope("ph2_probe"):
        copies = []
        for ch in range(NQ // 128):
            copies.append(pltpu.async_copy(
                table_hbm.at[qk.at[pl.ds(ch * 128, 128)]],
                lkp.at[pl.ds(ch * 128, 128)], sem))
        for cp in copies:
            cp.wait()

    scope3 = scope("ph3_compact")
    scope3.__enter__()
    wpos = jnp.int32(0)
    for i in range(K27):
        def c_body(g, w, i=i):
            f = i * Q + g * 16
            vals = lkp[pl.ds(f, 16)]
            hit = vals < SENT
            fidx = (i // 2) * NPAD + vals
            ab = (g * 16 + lane) * C * 2 + (i % 2)
            plsc.store_compressed(cfx.at[pl.ds(w, 16)], fidx, mask=hit)
            plsc.store_compressed(cab.at[pl.ds(w, 16)], ab, mask=hit)
            return w + jnp.sum(hit.astype(jnp.int32))
        wpos = lax.fori_loop(0, Q // 16, c_body, wpos)

    scope3.__exit__(None, None, None)

    def accum(buf, ch):
        for b in range(8):
            abv = cab[pl.ds(ch * 128 + b * 16, 16)]
            for j in range(16):
                a0 = abv[j]
                ab = lax.shift_right_arithmetic(a0, 1)
                cco = (a0 & 1) * C
                for k in range(4):
                    v = buf[b * 16 + j, pl.ds(cco + k * 16, 16)]
                    plsc.addupdate_scatter(outb, [ab + k * 16 + lane], v)

    def g_body(ch, _):
        with jax.named_scope("ph4_dma"):
            pltpu.async_copy(h_hbm.at[cfx.at[pl.ds(ch * 128, 128)]],
                             grows0, sem0).wait()
        with jax.named_scope("ph4_acc"):
            accum(grows0, ch)
        return 0
    nch = (wpos + 127) // 128
    with scope("ph4_gather_accum"):
        lax.fori_loop(0, nch, g_body, 0)

    with scope("ph5_out"):
        pltpu.sync_copy(outb, out_hbm.at[pl.ds(pbase * C, Q * C)])


@functools.lru_cache(maxsize=None)
def _sc_kernels():
    mesh = plsc.VectorSubcoreMesh(core_axis_name="c", subcore_axis_name="s")
    params = pltpu.CompilerParams(needs_layout_passes=False)
    build = pl.kernel(
        _sc_build,
        out_type=jax.ShapeDtypeStruct((TPAD,), jnp.int32),
        mesh=mesh,
        compiler_params=params,
        scratch_types=[
            pltpu.VMEM((3, NPAD), jnp.float32),
            pltpu.VMEM((TSH,), jnp.int32),
        ],
    )
    conv = pl.kernel(
        _sc_conv,
        out_type=jax.ShapeDtypeStruct((NPAD * C,), jnp.float32),
        mesh=mesh,
        compiler_params=params,
        scratch_types=[
            pltpu.VMEM((3, Q), jnp.float32),
            pltpu.VMEM((NQ,), jnp.int32),
            pltpu.VMEM((NQ,), jnp.int32),
            pltpu.VMEM((NQ_PAD,), jnp.int32),
            pltpu.VMEM((NQ_PAD,), jnp.int32),
            pltpu.VMEM((128, 2 * C), jnp.float32),
            pltpu.VMEM((128, 2 * C), jnp.float32),
            pltpu.VMEM((Q * C,), jnp.float32),
            pltpu.SemaphoreType.DMA,
            pltpu.SemaphoreType.DMA,
            pltpu.SemaphoreType.DMA,
        ],
    )
    return build, conv


def _h_block(f_ref, w_ref, o_ref):
    o_ref[...] = jnp.dot(f_ref[...], w_ref[...],
                         preferred_element_type=jnp.float32)


_PB = 1024


@jax.jit
def _tc_h(f_ext, w2):
    nb = NPAD // _PB
    return pl.pallas_call(
        _h_block,
        grid=(nb, HP),
        in_specs=[
            pl.BlockSpec((_PB, C), lambda b, m: (b, 0)),
            pl.BlockSpec((None, C, 2 * C), lambda b, m: (m, 0, 0)),
        ],
        out_specs=pl.BlockSpec((_PB, 2 * C), lambda b, m: (m * nb + b, 0)),
        out_shape=jax.ShapeDtypeStruct((HROWS, 2 * C), jnp.float32),
    )(f_ext, w2)


def kernel(features, in_positions, W):
    pos_t = jnp.zeros((3, NPAD), jnp.float32).at[:, :N].set(in_positions.T)
    f_ext = jnp.zeros((NPAD, C), jnp.float32).at[:N, :].set(features)
    w_pad = jnp.zeros((2 * HP, C, C), jnp.float32).at[:K27].set(W)
    w2 = w_pad.reshape(HP, 2, C, C).transpose(0, 2, 1, 3).reshape(HP, C, 2 * C)
    build, conv = _sc_kernels()
    table = build(pos_t)
    h = _tc_h(f_ext, w2)
    outf = conv(pos_t, table, h)
    return outf.reshape(NPAD, C)[:N]


# speedup vs baseline: 1.1541x; 1.1541x over previous
import functools

import jax
import jax.numpy as jnp
from jax import lax
from jax.experimental import pallas as pl
from jax.experimental.pallas import tpu as pltpu
from jax.experimental.pallas import tpu_sc as plsc

N = 20000
C = 64
K27 = 27
NPAD = 20480
Q = NPAD // 32
NGRP = N // 16
BASE = 68
TSH = 9840
TPAD = 32 * TSH
SENT = 2**30
NQ = K27 * Q
NQ_PAD = NQ + 128
HP = 14
HROWS = HP * NPAD

_LANE = lambda: lax.broadcasted_iota(jnp.int32, (16,), 0)


def _shift_right_one(v):
    idx = jnp.maximum(_LANE() - 1, 0)
    return lax.gather(
        v, idx[:, None],
        lax.GatherDimensionNumbers(
            offset_dims=(), collapsed_slice_dims=(0,), start_index_map=(0,)),
        slice_sizes=(1,),
        mode=lax.GatherScatterMode.PROMISE_IN_BOUNDS)


def _keys16(posb, g):
    off = g * 16
    x = posb[0, pl.ds(off, 16)].astype(jnp.int32)
    y = posb[1, pl.ds(off, 16)].astype(jnp.int32)
    z = posb[2, pl.ds(off, 16)].astype(jnp.int32)
    return ((x + 2) * BASE + (y + 2)) * BASE + (z + 2)


def _sc_build(pos_hbm, table_hbm, posb, tbl):
    wid = lax.axis_index("s") * 2 + lax.axis_index("c")
    shard_base = wid * TSH
    pltpu.sync_copy(pos_hbm, posb)

    def init_body(g, _):
        tbl[pl.ds(g * 16, 16)] = jnp.full((16,), SENT, jnp.int32)
        return 0
    lax.fori_loop(0, TSH // 16, init_body, 0)

    lane = _LANE()

    def body(g, _):
        key = _keys16(posb, g)
        p = g * 16 + lane
        dup = key != key
        for j in range(15):
            kj = key[j]
            dup = dup | ((lane > j) & (key == kj))
        addr = key - shard_base
        m = ~dup & (addr >= 0) & (addr < TSH)
        addr_c = jnp.clip(addr, 0, TSH - 1)
        cur = plsc.load_gather(tbl, [addr_c], mask=m)
        newv = jnp.minimum(jnp.where(m, cur, SENT), p)
        plsc.store_scatter(tbl, [addr_c], newv, mask=m)
        return 0
    lax.fori_loop(0, NGRP, body, 0)

    pltpu.sync_copy(tbl, table_hbm.at[pl.ds(shard_base, TSH)])


def _sc_conv(pos_hbm, table_hbm, h_hbm, out_hbm,
             posb, qk, lkp, cfx, cab, grows0, grows1, outb,
             sem, sem0, sem1):
    wid = lax.axis_index("s") * 2 + lax.axis_index("c")
    pbase = wid * Q
    pltpu.sync_copy(pos_hbm.at[:, pl.ds(pbase, Q)], posb)
    lane = _LANE()
    scope = jax.named_scope

    def zero_body(g, _):
        outb[pl.ds(g * 16, 16)] = jnp.zeros((16,), jnp.float32)
        return 0
    with scope("ph0_zero"):
        lax.fori_loop(0, Q * C // 16, zero_body, 0)

    def pad_body(g, _):
        f = g * 16 + lane
        cfx[pl.ds(g * 16, 16)] = (f % HP) * NPAD + (N + f % (NPAD - N))
        cab[pl.ds(g * 16, 16)] = jnp.full((16,), (Q * C - 1) * 2,
                                          jnp.int32)
        return 0
    with scope("ph0_pad"):
        lax.fori_loop(0, NQ_PAD // 16, pad_body, 0)

    scope1 = scope("ph1_keys")
    scope1.__enter__()
    for i in range(K27):
        dx, dy, dz = i // 9 - 1, (i // 3) % 3 - 1, i % 3 - 1
        doff = (dx * BASE + dy) * BASE + dz

        def q_body(g, _, i=i, doff=doff):
            key = _keys16(posb, g)
            qk[pl.ds(i * Q + g * 16, 16)] = key + doff
            return 0
        lax.fori_loop(0, Q // 16, q_body, 0)

    scope1.__exit__(None, None, None)

    with scope("ph2_probe"):
        copies = []
        for ch in range(NQ // 128):
            copies.append(pltpu.async_copy(
                table_hbm.at[qk.at[pl.ds(ch * 128, 128)]],
                lkp.at[pl.ds(ch * 128, 128)], sem))
        for cp in copies:
            cp.wait()

    scope3 = scope("ph3_compact")
    scope3.__enter__()
    wpos = jnp.int32(0)
    for i in range(K27):
        def c_body(g, w, i=i):
            f = i * Q + g * 16
            vals = lkp[pl.ds(f, 16)]
            hit = vals < SENT
            fidx = (i // 2) * NPAD + vals
            ab = (g * 16 + lane) * C * 2 + (i % 2)
            plsc.store_compressed(cfx.at[pl.ds(w, 16)], fidx, mask=hit)
            plsc.store_compressed(cab.at[pl.ds(w, 16)], ab, mask=hit)
            return w + jnp.sum(hit.astype(jnp.int32))
        wpos = lax.fori_loop(0, Q // 16, c_body, wpos)

    scope3.__exit__(None, None, None)

    def accum(buf, ch):
        for b in range(8):
            abv = cab[pl.ds(ch * 128 + b * 16, 16)]
            for j in range(16):
                a0 = abv[j]
                ab = lax.shift_right_arithmetic(a0, 1)
                cco = (a0 & 1) * C
                for k in range(4):
                    v = buf[b * 16 + j, pl.ds(cco + k * 16, 16)]
                    plsc.addupdate_scatter(outb, [ab + k * 16 + lane], v)

    def g_body(ch, _):
        with jax.named_scope("ph4_dma"):
            pltpu.async_copy(h_hbm.at[cfx.at[pl.ds(ch * 128, 128)]],
                             grows0, sem0).wait()
        with jax.named_scope("ph4_acc"):
            accum(grows0, ch)
        return 0
    nch = (wpos + 127) // 128
    with scope("ph4_gather_accum"):
        lax.fori_loop(0, nch, g_body, 0)

    with scope("ph5_out"):
        pltpu.sync_copy(outb, out_hbm.at[pl.ds(pbase * C, Q * C)])


@functools.lru_cache(maxsize=None)
def _sc_kernels():
    mesh = plsc.VectorSubcoreMesh(core_axis_name="c", subcore_axis_name="s")
    params = pltpu.CompilerParams(needs_layout_passes=False)
    build = pl.kernel(
        _sc_build,
        out_type=jax.ShapeDtypeStruct((TPAD,), jnp.int32),
        mesh=mesh,
        compiler_params=params,
        scratch_types=[
            pltpu.VMEM((3, NPAD), jnp.float32),
            pltpu.VMEM((TSH,), jnp.int32),
        ],
    )
    conv = pl.kernel(
        _sc_conv,
        out_type=jax.ShapeDtypeStruct((NPAD * C,), jnp.float32),
        mesh=mesh,
        compiler_params=params,
        scratch_types=[
            pltpu.VMEM((3, Q), jnp.float32),
            pltpu.VMEM((NQ,), jnp.int32),
            pltpu.VMEM((NQ,), jnp.int32),
            pltpu.VMEM((NQ_PAD,), jnp.int32),
            pltpu.VMEM((NQ_PAD,), jnp.int32),
            pltpu.VMEM((128, 2 * C), jnp.float32),
            pltpu.VMEM((128, 2 * C), jnp.float32),
            pltpu.VMEM((Q * C,), jnp.float32),
            pltpu.SemaphoreType.DMA,
            pltpu.SemaphoreType.DMA,
            pltpu.SemaphoreType.DMA,
        ],
    )
    return build, conv


def _h_block(f_ref, w_ref, o_ref):
    o_ref[...] = jnp.dot(f_ref[...], w_ref[...],
                         preferred_element_type=jnp.float32)


_PB = 1024


@jax.jit
def _tc_h(f_ext, w2):
    nb = NPAD // _PB
    return pl.pallas_call(
        _h_block,
        grid=(nb, HP),
        in_specs=[
            pl.BlockSpec((_PB, C), lambda b, m: (b, 0)),
            pl.BlockSpec((None, C, 2 * C), lambda b, m: (m, 0, 0)),
        ],
        out_specs=pl.BlockSpec((_PB, 2 * C), lambda b, m: (m * nb + b, 0)),
        out_shape=jax.ShapeDtypeStruct((HROWS, 2 * C), jnp.float32),
    )(f_ext, w2)


def kernel(features, in_positions, W):
    pos_t = jnp.zeros((3, NPAD), jnp.float32).at[:, :N].set(in_positions.T)
    f_ext = jnp.zeros((NPAD, C), jnp.float32).at[:N, :].set(features)
    w_pad = jnp.zeros((2 * HP, C, C), jnp.float32).at[:K27].set(W)
    w2 = w_pad.reshape(HP, 2, C, C).transpose(0, 2, 1, 3).reshape(HP, C, 2 * C)
    build, conv = _sc_kernels()
    table = build(pos_t)
    h = _tc_h(f_ext, w2)
    outf = conv(pos_t, table, h)
    return outf.reshape(NPAD, C)[:N]


# speedup vs baseline: 1.5143x; 1.3121x over previous
import functools

import jax
import jax.numpy as jnp
from jax import lax
from jax.experimental import pallas as pl
from jax.experimental.pallas import tpu as pltpu
from jax.experimental.pallas import tpu_sc as plsc

N = 20000
C = 64
K27 = 27
NPAD = 20480
Q = NPAD // 32
NGRP = N // 16
BASE = 68
TSH = 9840
TPAD = 32 * TSH
SENT = 2**30
NQ = K27 * Q
NQ_PAD = NQ + 128
HQ = 7
HROWS = HQ * NPAD

_LANE = lambda: lax.broadcasted_iota(jnp.int32, (16,), 0)


def _keys16(posb, g):
    off = g * 16
    x = posb[0, pl.ds(off, 16)].astype(jnp.int32)
    y = posb[1, pl.ds(off, 16)].astype(jnp.int32)
    z = posb[2, pl.ds(off, 16)].astype(jnp.int32)
    return ((x + 2) * BASE + (y + 2)) * BASE + (z + 2)


def _sc_build(pos_hbm, table_hbm, posb, tbl):
    wid = lax.axis_index("s") * 2 + lax.axis_index("c")
    shard_base = wid * TSH
    pltpu.sync_copy(pos_hbm, posb)

    def init_body(g, _):
        tbl[pl.ds(g * 16, 16)] = jnp.full((16,), SENT, jnp.int32)
        return 0
    lax.fori_loop(0, TSH // 16, init_body, 0)

    lane = _LANE()

    def body(g, _):
        key = _keys16(posb, g)
        p = g * 16 + lane
        dup = key != key
        for j in range(15):
            kj = key[j]
            dup = dup | ((lane > j) & (key == kj))
        addr = key - shard_base
        m = ~dup & (addr >= 0) & (addr < TSH)
        addr_c = jnp.clip(addr, 0, TSH - 1)
        cur = plsc.load_gather(tbl, [addr_c], mask=m)
        newv = jnp.minimum(jnp.where(m, cur, SENT), p)
        plsc.store_scatter(tbl, [addr_c], newv, mask=m)
        return 0
    lax.fori_loop(0, NGRP, body, 0)

    pltpu.sync_copy(tbl, table_hbm.at[pl.ds(shard_base, TSH)])


def _sc_conv(pos_hbm, table_hbm, h_hbm, out_hbm,
             posb, qk, lkp, cfx, cab, grows0, outb, sem, sem0):
    wid = lax.axis_index("s") * 2 + lax.axis_index("c")
    pbase = wid * Q
    pltpu.sync_copy(pos_hbm.at[:, pl.ds(pbase, Q)], posb)
    lane = _LANE()

    def zero_body(g, _):
        outb[pl.ds(g * 16, 16)] = jnp.zeros((16,), jnp.float32)
        return 0
    lax.fori_loop(0, Q * C // 16, zero_body, 0)

    def pad_body(g, _):
        f = g * 16 + lane
        cfx[pl.ds(g * 16, 16)] = (f % HQ) * NPAD + (N + f % (NPAD - N))
        cab[pl.ds(g * 16, 16)] = jnp.full((16,), (Q * C - 1) * 4,
                                          jnp.int32)
        return 0
    lax.fori_loop(0, NQ_PAD // 16, pad_body, 0)

    for i in range(K27):
        dx, dy, dz = i // 9 - 1, (i // 3) % 3 - 1, i % 3 - 1
        doff = (dx * BASE + dy) * BASE + dz

        def q_body(g, _, i=i, doff=doff):
            key = _keys16(posb, g)
            qk[pl.ds(i * Q + g * 16, 16)] = key + doff
            return 0
        lax.fori_loop(0, Q // 16, q_body, 0)

    copies = []
    for ch in range(NQ // 128):
        copies.append(pltpu.async_copy(
            table_hbm.at[qk.at[pl.ds(ch * 128, 128)]],
            lkp.at[pl.ds(ch * 128, 128)], sem))
    for cp in copies:
        cp.wait()

    wpos = jnp.int32(0)
    for i in range(K27):
        def c_body(g, w, i=i):
            f = i * Q + g * 16
            vals = lkp[pl.ds(f, 16)]
            hit = vals < SENT
            fidx = (i // 4) * NPAD + vals
            ab = ((g * 16 + lane) * C) * 4 + (i % 4)
            plsc.store_compressed(cfx.at[pl.ds(w, 16)], fidx, mask=hit)
            plsc.store_compressed(cab.at[pl.ds(w, 16)], ab, mask=hit)
            return w + jnp.sum(hit.astype(jnp.int32))
        wpos = lax.fori_loop(0, Q // 16, c_body, wpos)

    def accum(buf, ch):
        for b in range(4):
            abv = cab[pl.ds(ch * 64 + b * 16, 16)]
            for j in range(16):
                a0 = abv[j]
                ab = lax.shift_right_arithmetic(a0, 2)
                cco = (a0 & 3) * 32
                for k2 in range(2):
                    v32 = buf[b * 16 + j, pl.ds(cco + k2 * 16, 16)]
                    vbf = plsc.bitcast(v32, jnp.bfloat16)
                    va, vb = plsc.unpack(
                        vbf, format=plsc.PackFormat.INTERLEAVED)
                    plsc.addupdate_scatter(
                        outb, [ab + (2 * k2) * 16 + lane], va)
                    plsc.addupdate_scatter(
                        outb, [ab + (2 * k2 + 1) * 16 + lane], vb)

    def g_body(ch, _):
        pltpu.async_copy(h_hbm.at[cfx.at[pl.ds(ch * 64, 64)]],
                         grows0, sem0).wait()
        accum(grows0, ch)
        return 0
    nch = (wpos + 63) // 64
    lax.fori_loop(0, nch, g_body, 0)

    pltpu.sync_copy(outb, out_hbm.at[pl.ds(pbase * C, Q * C)])


@functools.lru_cache(maxsize=None)
def _sc_kernels():
    mesh = plsc.VectorSubcoreMesh(core_axis_name="c", subcore_axis_name="s")
    params = pltpu.CompilerParams(needs_layout_passes=False)
    build = pl.kernel(
        _sc_build,
        out_type=jax.ShapeDtypeStruct((TPAD,), jnp.int32),
        mesh=mesh,
        compiler_params=params,
        scratch_types=[
            pltpu.VMEM((3, NPAD), jnp.float32),
            pltpu.VMEM((TSH,), jnp.int32),
        ],
    )
    conv = pl.kernel(
        _sc_conv,
        out_type=jax.ShapeDtypeStruct((NPAD * C,), jnp.float32),
        mesh=mesh,
        compiler_params=params,
        scratch_types=[
            pltpu.VMEM((3, Q), jnp.float32),
            pltpu.VMEM((NQ,), jnp.int32),
            pltpu.VMEM((NQ,), jnp.int32),
            pltpu.VMEM((NQ_PAD,), jnp.int32),
            pltpu.VMEM((NQ_PAD,), jnp.int32),
            pltpu.VMEM((64, 2 * C), jnp.int32),
            pltpu.VMEM((Q * C,), jnp.float32),
            pltpu.SemaphoreType.DMA,
            pltpu.SemaphoreType.DMA,
        ],
    )
    return build, conv


def _h_block(f_ref, wa_ref, wb_ref, o_ref):
    oa = jnp.dot(f_ref[...], wa_ref[...], preferred_element_type=jnp.float32)
    ob = jnp.dot(f_ref[...], wb_ref[...], preferred_element_type=jnp.float32)
    lo = lax.bitcast_convert_type(oa.astype(jnp.bfloat16), jnp.uint16)
    hi = lax.bitcast_convert_type(ob.astype(jnp.bfloat16), jnp.uint16)
    packed = lo.astype(jnp.uint32) | (hi.astype(jnp.uint32) << 16)
    o_ref[...] = lax.bitcast_convert_type(packed, jnp.int32)


_PB = 1024


@jax.jit
def _tc_h(f_bf, wa, wb):
    nb = NPAD // _PB
    return pl.pallas_call(
        _h_block,
        grid=(nb, HQ),
        in_specs=[
            pl.BlockSpec((_PB, C), lambda b, m: (b, 0)),
            pl.BlockSpec((None, C, 2 * C), lambda b, m: (m, 0, 0)),
            pl.BlockSpec((None, C, 2 * C), lambda b, m: (m, 0, 0)),
        ],
        out_specs=pl.BlockSpec((_PB, 2 * C), lambda b, m: (m * nb + b, 0)),
        out_shape=jax.ShapeDtypeStruct((HROWS, 2 * C), jnp.int32),
    )(f_bf, wa, wb)


def kernel(features, in_positions, W):
    pos_t = jnp.zeros((3, NPAD), jnp.float32).at[:, :N].set(in_positions.T)
    f_ext = jnp.zeros((NPAD, C), jnp.float32).at[:N, :].set(features)
    w_pad = jnp.zeros((4 * HQ, C, C), jnp.float32).at[:K27].set(W)
    wq = w_pad.reshape(HQ, 4, C, C)
    cols_a = jnp.concatenate(
        [jnp.arange(16, dtype=jnp.int32), jnp.arange(32, 48, dtype=jnp.int32)])
    wa = wq[:, :, :, cols_a].transpose(0, 2, 1, 3).reshape(HQ, C, 2 * C)
    wb = wq[:, :, :, cols_a + 16].transpose(0, 2, 1, 3).reshape(HQ, C, 2 * C)
    f_bf = f_ext.astype(jnp.bfloat16)
    build, conv = _sc_kernels()
    table = build(pos_t)
    h = _tc_h(f_bf, wa.astype(jnp.bfloat16), wb.astype(jnp.bfloat16))
    outf = conv(pos_t, table, h)
    return outf.reshape(NPAD, C)[:N]


# speedup vs baseline: 1.6502x; 1.0898x over previous
import functools

import jax
import jax.numpy as jnp
from jax import lax
from jax.experimental import pallas as pl
from jax.experimental.pallas import tpu as pltpu
from jax.experimental.pallas import tpu_sc as plsc

N = 20000
C = 64
K27 = 27
NPAD = 20480
Q = NPAD // 32
NGRP = N // 16
BASE = 68
TSH = 9840
TPAD = 32 * TSH
SENT = 2**30
NQ = K27 * Q
NQ_PAD = NQ + 128
HQ = 7
HROWS = HQ * NPAD

_LANE = lambda: lax.broadcasted_iota(jnp.int32, (16,), 0)


def _keys16(posb, g):
    off = g * 16
    x = posb[0, pl.ds(off, 16)].astype(jnp.int32)
    y = posb[1, pl.ds(off, 16)].astype(jnp.int32)
    z = posb[2, pl.ds(off, 16)].astype(jnp.int32)
    return ((x + 2) * BASE + (y + 2)) * BASE + (z + 2)


def _sc_build(pos_hbm, table_hbm, posb, tbl):
    wid = lax.axis_index("s") * 2 + lax.axis_index("c")
    shard_base = wid * TSH
    pltpu.sync_copy(pos_hbm, posb)

    def init_body(g, _):
        tbl[pl.ds(g * 16, 16)] = jnp.full((16,), SENT, jnp.int32)
        return 0
    lax.fori_loop(0, TSH // 16, init_body, 0)

    lane = _LANE()

    def body(g, _):
        key = _keys16(posb, g)
        p = g * 16 + lane
        dup = key != key
        for j in range(15):
            kj = key[j]
            dup = dup | ((lane > j) & (key == kj))
        addr = key - shard_base
        m = ~dup & (addr >= 0) & (addr < TSH)
        addr_c = jnp.clip(addr, 0, TSH - 1)
        cur = plsc.load_gather(tbl, [addr_c], mask=m)
        newv = jnp.minimum(jnp.where(m, cur, SENT), p)
        plsc.store_scatter(tbl, [addr_c], newv, mask=m)
        return 0
    lax.fori_loop(0, NGRP, body, 0)

    pltpu.sync_copy(tbl, table_hbm.at[pl.ds(shard_base, TSH)])


def _sc_conv(pos_hbm, table_hbm, h_hbm, zer_hbm, padf_hbm, pada_hbm,
             out_hbm, posb, qk, lkp, cfx, cab, grows0, grows1, outb,
             sem, sem0, sem1, semi):
    wid = lax.axis_index("s") * 2 + lax.axis_index("c")
    pbase = wid * Q
    lane = _LANE()

    cp_pos = pltpu.async_copy(pos_hbm.at[:, pl.ds(pbase, Q)], posb, semi)
    pltpu.async_copy(zer_hbm, outb, sem0)
    pltpu.async_copy(padf_hbm, cfx, sem0)
    pltpu.async_copy(pada_hbm, cab, sem0)
    cp_pos.wait()

    for i in range(K27):
        dx, dy, dz = i // 9 - 1, (i // 3) % 3 - 1, i % 3 - 1
        doff = (dx * BASE + dy) * BASE + dz

        def q_body(g, _, i=i, doff=doff):
            key = _keys16(posb, g)
            qk[pl.ds(i * Q + g * 16, 16)] = key + doff
            return 0
        lax.fori_loop(0, Q // 16, q_body, 0)

    copies = []
    for ch in range(NQ // 128):
        copies.append(pltpu.async_copy(
            table_hbm.at[qk.at[pl.ds(ch * 128, 128)]],
            lkp.at[pl.ds(ch * 128, 128)], sem))
    for cp in copies:
        cp.wait()

    pltpu.make_async_copy(zer_hbm, outb, sem0).wait()
    pltpu.make_async_copy(padf_hbm, cfx, sem0).wait()
    pltpu.make_async_copy(pada_hbm, cab, sem0).wait()

    wpos = jnp.int32(0)
    for i in range(K27):
        def c_body(g, w, i=i):
            f = i * Q + g * 16
            vals = lkp[pl.ds(f, 16)]
            hit = vals < SENT
            fidx = (i // 4) * NPAD + vals
            ab = ((g * 16 + lane) * C) * 4 + (i % 4)
            plsc.store_compressed(cfx.at[pl.ds(w, 16)], fidx, mask=hit)
            plsc.store_compressed(cab.at[pl.ds(w, 16)], ab, mask=hit)
            return w + jnp.sum(hit.astype(jnp.int32))
        wpos = lax.fori_loop(0, Q // 16, c_body, wpos)

    def accum(buf, ch):
        for b in range(3):
            abv = cab[pl.ds(ch * 48 + b * 16, 16)]
            for j in range(16):
                a0 = abv[j]
                ab = lax.shift_right_arithmetic(a0, 2)
                cco = (a0 & 3) * 32
                for k2 in range(2):
                    v32 = buf[b * 16 + j, pl.ds(cco + k2 * 16, 16)]
                    vbf = plsc.bitcast(v32, jnp.bfloat16)
                    va, vb = plsc.unpack(
                        vbf, format=plsc.PackFormat.INTERLEAVED)
                    plsc.addupdate_scatter(
                        outb, [ab + (2 * k2) * 16 + lane], va)
                    plsc.addupdate_scatter(
                        outb, [ab + (2 * k2 + 1) * 16 + lane], vb)

    def start(ch, buf, s_):
        pltpu.async_copy(h_hbm.at[cfx.at[pl.ds(ch * 48, 48)]], buf, s_)

    def drain(buf, s_):
        pltpu.make_async_copy(h_hbm.at[pl.ds(0, 48)], buf, s_).wait()

    nch2 = ((wpos + 47) // 48 + 1) // 2
    start(0, grows0, sem0)

    def g_body(c2, _):
        start(2 * c2 + 1, grows1, sem1)
        drain(grows0, sem0)
        accum(grows0, 2 * c2)
        start(2 * c2 + 2, grows0, sem0)
        drain(grows1, sem1)
        accum(grows1, 2 * c2 + 1)
        return 0
    lax.fori_loop(0, nch2, g_body, 0)
    drain(grows0, sem0)

    pltpu.sync_copy(outb, out_hbm.at[pl.ds(pbase * C, Q * C)])


@functools.lru_cache(maxsize=None)
def _sc_kernels():
    mesh = plsc.VectorSubcoreMesh(core_axis_name="c", subcore_axis_name="s")
    params = pltpu.CompilerParams(needs_layout_passes=False)
    build = pl.kernel(
        _sc_build,
        out_type=jax.ShapeDtypeStruct((TPAD,), jnp.int32),
        mesh=mesh,
        compiler_params=params,
        scratch_types=[
            pltpu.VMEM((3, NPAD), jnp.float32),
            pltpu.VMEM((TSH,), jnp.int32),
        ],
    )
    conv = pl.kernel(
        _sc_conv,
        out_type=jax.ShapeDtypeStruct((NPAD * C,), jnp.float32),
        mesh=mesh,
        compiler_params=params,
        scratch_types=[
            pltpu.VMEM((3, Q), jnp.float32),
            pltpu.VMEM((NQ,), jnp.int32),
            pltpu.VMEM((NQ,), jnp.int32),
            pltpu.VMEM((NQ_PAD,), jnp.int32),
            pltpu.VMEM((NQ_PAD,), jnp.int32),
            pltpu.VMEM((48, 2 * C), jnp.int32),
            pltpu.VMEM((48, 2 * C), jnp.int32),
            pltpu.VMEM((Q * C,), jnp.float32),
            pltpu.SemaphoreType.DMA,
            pltpu.SemaphoreType.DMA,
            pltpu.SemaphoreType.DMA,
            pltpu.SemaphoreType.DMA,
        ],
    )
    return build, conv


def _h_block(f_ref, wa_ref, wb_ref, o_ref):
    oa = jnp.dot(f_ref[...], wa_ref[...], preferred_element_type=jnp.float32)
    ob = jnp.dot(f_ref[...], wb_ref[...], preferred_element_type=jnp.float32)
    lo = lax.bitcast_convert_type(oa.astype(jnp.bfloat16), jnp.uint16)
    hi = lax.bitcast_convert_type(ob.astype(jnp.bfloat16), jnp.uint16)
    packed = lo.astype(jnp.uint32) | (hi.astype(jnp.uint32) << 16)
    o_ref[...] = lax.bitcast_convert_type(packed, jnp.int32)


_PB = 1024


@jax.jit
def _tc_h(f_bf, wa, wb):
    nb = NPAD // _PB
    return pl.pallas_call(
        _h_block,
        grid=(nb, HQ),
        in_specs=[
            pl.BlockSpec((_PB, C), lambda b, m: (b, 0)),
            pl.BlockSpec((None, C, 2 * C), lambda b, m: (m, 0, 0)),
            pl.BlockSpec((None, C, 2 * C), lambda b, m: (m, 0, 0)),
        ],
        out_specs=pl.BlockSpec((_PB, 2 * C), lambda b, m: (m * nb + b, 0)),
        out_shape=jax.ShapeDtypeStruct((HROWS, 2 * C), jnp.int32),
    )(f_bf, wa, wb)


def kernel(features, in_positions, W):
    pos_t = jnp.zeros((3, NPAD), jnp.float32).at[:, :N].set(in_positions.T)
    f_ext = jnp.zeros((NPAD, C), jnp.float32).at[:N, :].set(features)
    w_pad = jnp.zeros((4 * HQ, C, C), jnp.float32).at[:K27].set(W)
    wq = w_pad.reshape(HQ, 4, C, C)
    cols_a = jnp.concatenate(
        [jnp.arange(16, dtype=jnp.int32), jnp.arange(32, 48, dtype=jnp.int32)])
    wa = wq[:, :, :, cols_a].transpose(0, 2, 1, 3).reshape(HQ, C, 2 * C)
    wb = wq[:, :, :, cols_a + 16].transpose(0, 2, 1, 3).reshape(HQ, C, 2 * C)
    f_bf = f_ext.astype(jnp.bfloat16)
    build, conv = _sc_kernels()
    table = build(pos_t)
    h = _tc_h(f_bf, wa.astype(jnp.bfloat16), wb.astype(jnp.bfloat16))
    zer = jnp.zeros((Q * C,), jnp.float32)
    fidx = jnp.arange(NQ_PAD, dtype=jnp.int32)
    padf = (fidx % HQ) * NPAD + (N + fidx % (NPAD - N))
    pada = jnp.full((NQ_PAD,), (Q * C - 1) * 4, jnp.int32)
    outf = conv(pos_t, table, h, zer, padf, pada)
    return outf.reshape(NPAD, C)[:N]


# speedup vs baseline: 1.6596x; 1.0057x over previous
import functools

import jax
import jax.numpy as jnp
from jax import lax
from jax.experimental import pallas as pl
from jax.experimental.pallas import tpu as pltpu
from jax.experimental.pallas import tpu_sc as plsc

N = 20000
C = 64
K27 = 27
NPAD = 20480
Q = NPAD // 32
NGRP = N // 16
BASE = 68
TSH = 9840
TPAD = 32 * TSH
SENT = 2**30
NQ = K27 * Q
NQ_PAD = NQ + 128
HQ = 7
HROWS = HQ * NPAD

_LANE = lambda: lax.broadcasted_iota(jnp.int32, (16,), 0)


def _keys16(posb, g):
    off = g * 16
    x = posb[0, pl.ds(off, 16)].astype(jnp.int32)
    y = posb[1, pl.ds(off, 16)].astype(jnp.int32)
    z = posb[2, pl.ds(off, 16)].astype(jnp.int32)
    return ((x + 2) * BASE + (y + 2)) * BASE + (z + 2)


def _sc_build(pos_hbm, table_hbm, posb, tbl, losers):
    wid = lax.axis_index("s") * 2 + lax.axis_index("c")
    shard_base = wid * TSH
    pltpu.sync_copy(pos_hbm, posb)

    def init_body(g, _):
        tbl[pl.ds(g * 16, 16)] = jnp.full((16,), SENT, jnp.int32)
        return 0
    lax.fori_loop(0, TSH // 16, init_body, 0)

    lane = _LANE()

    def body_a(g, _):
        key = _keys16(posb, g)
        addr = key - shard_base
        m = (addr >= 0) & (addr < TSH)
        addr_c = jnp.clip(addr, 0, TSH - 1)
        plsc.store_scatter(tbl, [addr_c], g * 16 + lane, mask=m)
        return 0
    lax.fori_loop(0, NGRP, body_a, 0)

    def body_b(g, w):
        addr = _keys16(posb, g) - shard_base
        m = (addr >= 0) & (addr < TSH)
        addr_c = jnp.clip(addr, 0, TSH - 1)
        p = g * 16 + lane
        cur = plsc.load_gather(tbl, [addr_c], mask=m)
        lose = m & (cur != p)
        plsc.store_compressed(losers.at[pl.ds(w, 16)],
                              addr_c * 32768 + p, mask=lose)
        return w + jnp.sum(lose.astype(jnp.int32))
    nl = lax.fori_loop(0, NGRP, body_b, 0)

    losers[pl.ds(nl, 16)] = jnp.full((16,), (TSH - 1) * 32768 + 32767,
                                     jnp.int32)

    def body_c(g, _):
        enc = losers[pl.ds(g * 16, 16)]
        a = lax.shift_right_arithmetic(enc, 15)
        p = enc & 32767
        dup = a != a
        for j in range(15):
            aj = a[j]
            dup = dup | ((lane > j) & (a == aj))
        m2 = ~dup & (p != 32767)
        cur = plsc.load_gather(tbl, [a], mask=m2)
        newv = jnp.minimum(jnp.where(m2, cur, SENT), p)
        plsc.store_scatter(tbl, [a], newv, mask=m2)
        return 0
    lax.fori_loop(0, (nl + 15) // 16, body_c, 0)

    pltpu.sync_copy(tbl, table_hbm.at[pl.ds(shard_base, TSH)])


def _sc_conv(pos_hbm, table_hbm, h_hbm, zer_hbm, padf_hbm, pada_hbm,
             out_hbm, posb, qk, lkp, cfx, cab, grows0, grows1, outb,
             sem, sem0, sem1, semi):
    wid = lax.axis_index("s") * 2 + lax.axis_index("c")
    pbase = wid * Q
    lane = _LANE()

    cp_pos = pltpu.async_copy(pos_hbm.at[:, pl.ds(pbase, Q)], posb, semi)
    pltpu.async_copy(zer_hbm, outb, sem0)
    pltpu.async_copy(padf_hbm, cfx, sem0)
    pltpu.async_copy(pada_hbm, cab, sem0)
    cp_pos.wait()

    for i in range(K27):
        dx, dy, dz = i // 9 - 1, (i // 3) % 3 - 1, i % 3 - 1
        doff = (dx * BASE + dy) * BASE + dz

        def q_body(g, _, i=i, doff=doff):
            key = _keys16(posb, g)
            qk[pl.ds(i * Q + g * 16, 16)] = key + doff
            return 0
        lax.fori_loop(0, Q // 16, q_body, 0)

    copies = []
    for ch in range(NQ // 128):
        copies.append(pltpu.async_copy(
            table_hbm.at[qk.at[pl.ds(ch * 128, 128)]],
            lkp.at[pl.ds(ch * 128, 128)], sem))
    for cp in copies:
        cp.wait()

    pltpu.make_async_copy(zer_hbm, outb, sem0).wait()
    pltpu.make_async_copy(padf_hbm, cfx, sem0).wait()
    pltpu.make_async_copy(pada_hbm, cab, sem0).wait()

    wpos = jnp.int32(0)
    for i in range(K27):
        def c_body(g, w, i=i):
            f = i * Q + g * 16
            vals = lkp[pl.ds(f, 16)]
            hit = vals < SENT
            fidx = (i // 4) * NPAD + vals
            ab = ((g * 16 + lane) * C) * 4 + (i % 4)
            plsc.store_compressed(cfx.at[pl.ds(w, 16)], fidx, mask=hit)
            plsc.store_compressed(cab.at[pl.ds(w, 16)], ab, mask=hit)
            return w + jnp.sum(hit.astype(jnp.int32))
        wpos = lax.fori_loop(0, Q // 16, c_body, wpos)

    def accum(buf, ch):
        for b in range(3):
            abv = cab[pl.ds(ch * 48 + b * 16, 16)]
            for j in range(16):
                a0 = abv[j]
                ab = lax.shift_right_arithmetic(a0, 2)
                cco = (a0 & 3) * 32
                for k2 in range(2):
                    v32 = buf[b * 16 + j, pl.ds(cco + k2 * 16, 16)]
                    vbf = plsc.bitcast(v32, jnp.bfloat16)
                    va, vb = plsc.unpack(
                        vbf, format=plsc.PackFormat.INTERLEAVED)
                    plsc.addupdate_scatter(
                        outb, [ab + (2 * k2) * 16 + lane], va)
                    plsc.addupdate_scatter(
                        outb, [ab + (2 * k2 + 1) * 16 + lane], vb)

    def start(ch, buf, s_):
        pltpu.async_copy(h_hbm.at[cfx.at[pl.ds(ch * 48, 48)]], buf, s_)

    def drain(buf, s_):
        pltpu.make_async_copy(h_hbm.at[pl.ds(0, 48)], buf, s_).wait()

    nch2 = ((wpos + 47) // 48 + 1) // 2
    start(0, grows0, sem0)

    def g_body(c2, _):
        start(2 * c2 + 1, grows1, sem1)
        drain(grows0, sem0)
        accum(grows0, 2 * c2)
        start(2 * c2 + 2, grows0, sem0)
        drain(grows1, sem1)
        accum(grows1, 2 * c2 + 1)
        return 0
    lax.fori_loop(0, nch2, g_body, 0)
    drain(grows0, sem0)

    pltpu.sync_copy(outb, out_hbm.at[pl.ds(pbase * C, Q * C)])


@functools.lru_cache(maxsize=None)
def _sc_kernels():
    mesh = plsc.VectorSubcoreMesh(core_axis_name="c", subcore_axis_name="s")
    params = pltpu.CompilerParams(needs_layout_passes=False)
    build = pl.kernel(
        _sc_build,
        out_type=jax.ShapeDtypeStruct((TPAD,), jnp.int32),
        mesh=mesh,
        compiler_params=params,
        scratch_types=[
            pltpu.VMEM((3, NPAD), jnp.float32),
            pltpu.VMEM((TSH,), jnp.int32),
            pltpu.VMEM((N + 16,), jnp.int32),
        ],
    )
    conv = pl.kernel(
        _sc_conv,
        out_type=jax.ShapeDtypeStruct((NPAD * C,), jnp.float32),
        mesh=mesh,
        compiler_params=params,
        scratch_types=[
            pltpu.VMEM((3, Q), jnp.float32),
            pltpu.VMEM((NQ,), jnp.int32),
            pltpu.VMEM((NQ,), jnp.int32),
            pltpu.VMEM((NQ_PAD,), jnp.int32),
            pltpu.VMEM((NQ_PAD,), jnp.int32),
            pltpu.VMEM((48, 2 * C), jnp.int32),
            pltpu.VMEM((48, 2 * C), jnp.int32),
            pltpu.VMEM((Q * C,), jnp.float32),
            pltpu.SemaphoreType.DMA,
            pltpu.SemaphoreType.DMA,
            pltpu.SemaphoreType.DMA,
            pltpu.SemaphoreType.DMA,
        ],
    )
    return build, conv


def _h_block(f_ref, wa_ref, wb_ref, o_ref):
    oa = jnp.dot(f_ref[...], wa_ref[...], preferred_element_type=jnp.float32)
    ob = jnp.dot(f_ref[...], wb_ref[...], preferred_element_type=jnp.float32)
    lo = lax.bitcast_convert_type(oa.astype(jnp.bfloat16), jnp.uint16)
    hi = lax.bitcast_convert_type(ob.astype(jnp.bfloat16), jnp.uint16)
    packed = lo.astype(jnp.uint32) | (hi.astype(jnp.uint32) << 16)
    o_ref[...] = lax.bitcast_convert_type(packed, jnp.int32)


_PB = 1024


@jax.jit
def _tc_h(f_bf, wa, wb):
    nb = NPAD // _PB
    return pl.pallas_call(
        _h_block,
        grid=(nb, HQ),
        in_specs=[
            pl.BlockSpec((_PB, C), lambda b, m: (b, 0)),
            pl.BlockSpec((None, C, 2 * C), lambda b, m: (m, 0, 0)),
            pl.BlockSpec((None, C, 2 * C), lambda b, m: (m, 0, 0)),
        ],
        out_specs=pl.BlockSpec((_PB, 2 * C), lambda b, m: (m * nb + b, 0)),
        out_shape=jax.ShapeDtypeStruct((HROWS, 2 * C), jnp.int32),
    )(f_bf, wa, wb)


def kernel(features, in_positions, W):
    pos_t = jnp.zeros((3, NPAD), jnp.float32).at[:, :N].set(in_positions.T)
    f_ext = jnp.zeros((NPAD, C), jnp.float32).at[:N, :].set(features)
    w_pad = jnp.zeros((4 * HQ, C, C), jnp.float32).at[:K27].set(W)
    wq = w_pad.reshape(HQ, 4, C, C)
    cols_a = jnp.concatenate(
        [jnp.arange(16, dtype=jnp.int32), jnp.arange(32, 48, dtype=jnp.int32)])
    wa = wq[:, :, :, cols_a].transpose(0, 2, 1, 3).reshape(HQ, C, 2 * C)
    wb = wq[:, :, :, cols_a + 16].transpose(0, 2, 1, 3).reshape(HQ, C, 2 * C)
    f_bf = f_ext.astype(jnp.bfloat16)
    build, conv = _sc_kernels()
    table = build(pos_t)
    h = _tc_h(f_bf, wa.astype(jnp.bfloat16), wb.astype(jnp.bfloat16))
    zer = jnp.zeros((Q * C,), jnp.float32)
    fidx = jnp.arange(NQ_PAD, dtype=jnp.int32)
    padf = (fidx % HQ) * NPAD + (N + fidx % (NPAD - N))
    pada = jnp.full((NQ_PAD,), (Q * C - 1) * 4, jnp.int32)
    outf = conv(pos_t, table, h, zer, padf, pada)
    return outf.reshape(NPAD, C)[:N]
